# baseline (speedup 1.0000x reference)
import functools

import jax
import jax.numpy as jnp
from jax import lax
from jax.experimental import pallas as pl
from jax.experimental.pallas import tpu as pltpu
from jax.experimental.pallas import tpu_sc as plsc

BATCH = 16384
EMBED = 64
NUM_LABELS = 16
VOCAB = 1000000
NUM_WORKERS = 32
BPW = BATCH // NUM_WORKERS
CHUNK = 128
NCHUNK = BPW // CHUNK
LANES = 16

TILE_I = 512
GRID = (VOCAB + TILE_I - 1) // TILE_I
NBLK = (VOCAB + CHUNK - 1) // CHUNK
MROWS = NBLK * NUM_LABELS

_mesh = plsc.VectorSubcoreMesh(core_axis_name="c", subcore_axis_name="s")


def _mm_body(l_ref, wt_ref, out_ref):
    p = lax.dot_general(l_ref[...], wt_ref[...],
                        (((1,), (0,)), ((), ())),
                        preferred_element_type=jnp.float32)
    out_ref[...] = (
        p.reshape(NUM_LABELS, TILE_I // CHUNK, CHUNK)
        .transpose(1, 0, 2)
        .reshape(TILE_I // CHUNK * NUM_LABELS, CHUNK))


_mm = pl.pallas_call(
    _mm_body,
    out_shape=jax.ShapeDtypeStruct((MROWS, CHUNK), jnp.float32),
    grid=(GRID,),
    in_specs=[
        pl.BlockSpec((NUM_LABELS, EMBED), lambda b: (0, 0)),
        pl.BlockSpec((EMBED, TILE_I), lambda b: (0, b)),
    ],
    out_specs=pl.BlockSpec((TILE_I // CHUNK * NUM_LABELS, CHUNK),
                           lambda b: (b, 0)),
)


@functools.partial(
    pl.kernel,
    out_type=jax.ShapeDtypeStruct((BATCH,), jnp.float32),
    mesh=_mesh,
    compiler_params=pltpu.CompilerParams(needs_layout_passes=False,
                                         use_tc_tiling_on_sc=True),
    scratch_types=[
        pltpu.VMEM((NCHUNK, CHUNK), jnp.int32),
        pltpu.VMEM((NCHUNK, CHUNK), jnp.int32),
        pltpu.VMEM((1, BPW), jnp.int32),
        pltpu.VMEM((BPW, CHUNK), jnp.float32),
        pltpu.VMEM((BPW,), jnp.float32),
        pltpu.SemaphoreType.DMA,
    ],
)
def _extract(dataT_hbm, mf_hbm, out_hbm,
             idx_v, ridx_v, lbl_v, rows_v, out_v, sem):
    wid = lax.axis_index("s") * 2 + lax.axis_index("c")
    base = wid * BPW

    for c in range(NCHUNK):
        pltpu.sync_copy(
            dataT_hbm.at[pl.ds(0, 1), pl.ds(base + c * CHUNK, CHUNK)],
            idx_v.at[pl.ds(c, 1)])
    pltpu.sync_copy(dataT_hbm.at[pl.ds(1, 1), pl.ds(base, BPW)], lbl_v)
    for c in range(NCHUNK):
        for g in range(CHUNK // LANES):
            w = idx_v[c, pl.ds(g * LANES, LANES)]
            l = lbl_v.at[0][pl.ds(c * CHUNK + g * LANES, LANES)]
            ridx_v[c, pl.ds(g * LANES, LANES)] = ((w >> 7) << 4) + l
    copies = [
        pltpu.async_copy(
            mf_hbm.at[ridx_v.at[c]],
            rows_v.at[pl.ds(c * CHUNK, CHUNK)],
            sem,
        )
        for c in range(NCHUNK)
    ]
    for cp in copies:
        cp.wait()

    iota = lax.iota(jnp.int32, LANES)

    def group(g, carry):
        gbase = pl.multiple_of(g * LANES, LANES)
        cvec = idx_v.at[g // (CHUNK // LANES)][
            pl.ds((g % (CHUNK // LANES)) * LANES, LANES)] & 127
        acc = jnp.zeros((LANES,), jnp.float32)
        for j in range(LANES):
            cj = cvec[j]
            v = rows_v.at[gbase + j][pl.ds((cj >> 4) << 4, LANES)]
            sel = v.at[jnp.full((LANES,), cj & 15, jnp.int32)].get(
                mode="promise_in_bounds")
            acc = jnp.where(iota == j, sel, acc)
        out_v[pl.ds(gbase, LANES)] = acc
        return carry

    lax.fori_loop(0, BPW // LANES, group, 0)

    pltpu.sync_copy(out_v, out_hbm.at[pl.ds(base, BPW)])


def kernel(data, target, word_embedding, label_embedding):
    del target
    mf = _mm(label_embedding, word_embedding.T)
    return _extract(data.astype(jnp.int32).T, mf)


# speedup vs baseline: 6.8787x; 6.8787x over previous
import functools

import jax
import jax.numpy as jnp
from jax import lax
from jax.experimental import pallas as pl
from jax.experimental.pallas import tpu as pltpu
from jax.experimental.pallas import tpu_sc as plsc

BATCH = 16384
EMBED = 64
NUM_LABELS = 16
VOCAB = 1000000
NUM_WORKERS = 32
BPW = BATCH // NUM_WORKERS
CHUNK = 128
NCHUNK = BPW // CHUNK
LANES = 16

TILE_I = 8192
GRID = (VOCAB + TILE_I - 1) // TILE_I
NBLK = (VOCAB + CHUNK - 1) // CHUNK
MROWS = NBLK * NUM_LABELS

_mesh = plsc.VectorSubcoreMesh(core_axis_name="c", subcore_axis_name="s")


def _mm_body(l_ref, wt_ref, out_ref):
    p = lax.dot_general(l_ref[...], wt_ref[...],
                        (((1,), (0,)), ((), ())),
                        preferred_element_type=jnp.float32)
    out_ref[...] = (
        p.reshape(NUM_LABELS, TILE_I // CHUNK, CHUNK)
        .transpose(1, 0, 2)
        .reshape(TILE_I // CHUNK * NUM_LABELS, CHUNK))


_mm = pl.pallas_call(
    _mm_body,
    out_shape=jax.ShapeDtypeStruct((MROWS, CHUNK), jnp.float32),
    grid=(GRID,),
    in_specs=[
        pl.BlockSpec((NUM_LABELS, EMBED), lambda b: (0, 0)),
        pl.BlockSpec((EMBED, TILE_I), lambda b: (0, b)),
    ],
    out_specs=pl.BlockSpec((TILE_I // CHUNK * NUM_LABELS, CHUNK),
                           lambda b: (b, 0)),
)


@functools.partial(
    pl.kernel,
    out_type=jax.ShapeDtypeStruct((BATCH,), jnp.float32),
    mesh=_mesh,
    compiler_params=pltpu.CompilerParams(needs_layout_passes=False,
                                         use_tc_tiling_on_sc=True),
    scratch_types=[
        pltpu.VMEM((NCHUNK, CHUNK), jnp.int32),
        pltpu.VMEM((NCHUNK, CHUNK), jnp.int32),
        pltpu.VMEM((1, BPW), jnp.int32),
        pltpu.VMEM((BPW, CHUNK), jnp.float32),
        pltpu.VMEM((BPW,), jnp.float32),
        pltpu.SemaphoreType.DMA,
    ],
)
def _extract(dataT_hbm, mf_hbm, out_hbm,
             idx_v, ridx_v, lbl_v, rows_v, out_v, sem):
    wid = lax.axis_index("s") * 2 + lax.axis_index("c")
    base = wid * BPW

    for c in range(NCHUNK):
        pltpu.sync_copy(
            dataT_hbm.at[pl.ds(0, 1), pl.ds(base + c * CHUNK, CHUNK)],
            idx_v.at[pl.ds(c, 1)])
    pltpu.sync_copy(dataT_hbm.at[pl.ds(1, 1), pl.ds(base, BPW)], lbl_v)
    for c in range(NCHUNK):
        for g in range(CHUNK // LANES):
            w = idx_v[c, pl.ds(g * LANES, LANES)]
            l = lbl_v.at[0][pl.ds(c * CHUNK + g * LANES, LANES)]
            ridx_v[c, pl.ds(g * LANES, LANES)] = ((w >> 7) << 4) + l
    copies = [
        pltpu.async_copy(
            mf_hbm.at[ridx_v.at[c]],
            rows_v.at[pl.ds(c * CHUNK, CHUNK)],
            sem,
        )
        for c in range(NCHUNK)
    ]
    for cp in copies:
        cp.wait()

    iota = lax.iota(jnp.int32, LANES)

    def group(g, carry):
        gbase = pl.multiple_of(g * LANES, LANES)
        cvec = idx_v.at[g // (CHUNK // LANES)][
            pl.ds((g % (CHUNK // LANES)) * LANES, LANES)] & 127
        acc = jnp.zeros((LANES,), jnp.float32)
        for j in range(LANES):
            cj = cvec[j]
            v = rows_v.at[gbase + j][pl.ds((cj >> 4) << 4, LANES)]
            sel = v.at[jnp.full((LANES,), cj & 15, jnp.int32)].get(
                mode="promise_in_bounds")
            acc = jnp.where(iota == j, sel, acc)
        out_v[pl.ds(gbase, LANES)] = acc
        return carry

    lax.fori_loop(0, BPW // LANES, group, 0)

    pltpu.sync_copy(out_v, out_hbm.at[pl.ds(base, BPW)])


def kernel(data, target, word_embedding, label_embedding):
    del target
    mf = _mm(label_embedding, word_embedding.T)
    return _extract(data.astype(jnp.int32).T, mf)


# speedup vs baseline: 8.6817x; 1.2621x over previous
import functools

import jax
import jax.numpy as jnp
from jax import lax
from jax.experimental import pallas as pl
from jax.experimental.pallas import tpu as pltpu
from jax.experimental.pallas import tpu_sc as plsc

BATCH = 16384
EMBED = 64
NUM_LABELS = 16
VOCAB = 1000000
NUM_WORKERS = 32
BPW = BATCH // NUM_WORKERS
CHUNK = 128
NCHUNK = BPW // CHUNK
LANES = 16

TILE_I = 16384
GRID = (VOCAB + TILE_I - 1) // TILE_I
NBLK = (VOCAB + CHUNK - 1) // CHUNK
MROWS = NBLK * NUM_LABELS

_mesh = plsc.VectorSubcoreMesh(core_axis_name="c", subcore_axis_name="s")


def _mm_body(l_ref, wt_ref, out_ref):
    p = lax.dot_general(l_ref[...], wt_ref[...],
                        (((1,), (0,)), ((), ())),
                        preferred_element_type=jnp.float32)
    out_ref[...] = (
        p.reshape(NUM_LABELS, TILE_I // CHUNK, CHUNK)
        .transpose(1, 0, 2)
        .reshape(TILE_I // CHUNK * NUM_LABELS, CHUNK))


_mm = pl.pallas_call(
    _mm_body,
    out_shape=jax.ShapeDtypeStruct((MROWS, CHUNK), jnp.float32),
    grid=(GRID,),
    in_specs=[
        pl.BlockSpec((NUM_LABELS, EMBED), lambda b: (0, 0)),
        pl.BlockSpec((EMBED, TILE_I), lambda b: (0, b)),
    ],
    out_specs=pl.BlockSpec((TILE_I // CHUNK * NUM_LABELS, CHUNK),
                           lambda b: (b, 0)),
)


@functools.partial(
    pl.kernel,
    out_type=jax.ShapeDtypeStruct((BATCH,), jnp.float32),
    mesh=_mesh,
    compiler_params=pltpu.CompilerParams(needs_layout_passes=False,
                                         use_tc_tiling_on_sc=True),
    scratch_types=[
        pltpu.VMEM((NCHUNK, CHUNK), jnp.int32),
        pltpu.VMEM((NCHUNK, CHUNK), jnp.int32),
        pltpu.VMEM((1, BPW), jnp.int32),
        pltpu.VMEM((BPW, CHUNK), jnp.float32),
        pltpu.VMEM((BPW,), jnp.float32),
        pltpu.SemaphoreType.DMA,
    ],
)
def _extract(dataT_hbm, mf_hbm, out_hbm,
             idx_v, ridx_v, lbl_v, rows_v, out_v, sem):
    wid = lax.axis_index("s") * 2 + lax.axis_index("c")
    base = wid * BPW

    for c in range(NCHUNK):
        pltpu.sync_copy(
            dataT_hbm.at[pl.ds(0, 1), pl.ds(base + c * CHUNK, CHUNK)],
            idx_v.at[pl.ds(c, 1)])
    pltpu.sync_copy(dataT_hbm.at[pl.ds(1, 1), pl.ds(base, BPW)], lbl_v)
    for c in range(NCHUNK):
        for g in range(CHUNK // LANES):
            w = idx_v[c, pl.ds(g * LANES, LANES)]
            l = lbl_v.at[0][pl.ds(c * CHUNK + g * LANES, LANES)]
            ridx_v[c, pl.ds(g * LANES, LANES)] = ((w >> 7) << 4) + l
    copies = [
        pltpu.async_copy(
            mf_hbm.at[ridx_v.at[c]],
            rows_v.at[pl.ds(c * CHUNK, CHUNK)],
            sem,
        )
        for c in range(NCHUNK)
    ]
    for cp in copies:
        cp.wait()

    iota = lax.iota(jnp.int32, LANES)

    def group(g, carry):
        gbase = pl.multiple_of(g * LANES, LANES)
        cvec = idx_v.at[g // (CHUNK // LANES)][
            pl.ds((g % (CHUNK // LANES)) * LANES, LANES)] & 127
        acc = jnp.zeros((LANES,), jnp.float32)
        for j in range(LANES):
            cj = cvec[j]
            v = rows_v.at[gbase + j][pl.ds((cj >> 4) << 4, LANES)]
            sel = v.at[jnp.full((LANES,), cj & 15, jnp.int32)].get(
                mode="promise_in_bounds")
            acc = jnp.where(iota == j, sel, acc)
        out_v[pl.ds(gbase, LANES)] = acc
        return carry

    lax.fori_loop(0, BPW // LANES, group, 0)

    pltpu.sync_copy(out_v, out_hbm.at[pl.ds(base, BPW)])


def kernel(data, target, word_embedding, label_embedding):
    del target
    mf = _mm(label_embedding, word_embedding.T)
    return _extract(data.astype(jnp.int32).T, mf)


# speedup vs baseline: 9.3459x; 1.0765x over previous
import functools

import jax
import jax.numpy as jnp
from jax import lax
from jax.experimental import pallas as pl
from jax.experimental.pallas import tpu as pltpu
from jax.experimental.pallas import tpu_sc as plsc

BATCH = 16384
EMBED = 64
NUM_LABELS = 16
VOCAB = 1000000
NUM_WORKERS = 32
BPW = BATCH // NUM_WORKERS
CHUNK = 128
NCHUNK = BPW // CHUNK
LANES = 16

TILE_I = 32768
GRID = (VOCAB + TILE_I - 1) // TILE_I
NBLK = (VOCAB + CHUNK - 1) // CHUNK
MROWS = NBLK * NUM_LABELS

_mesh = plsc.VectorSubcoreMesh(core_axis_name="c", subcore_axis_name="s")


def _mm_body(l_ref, wt_ref, out_ref):
    p = lax.dot_general(l_ref[...], wt_ref[...],
                        (((1,), (0,)), ((), ())),
                        preferred_element_type=jnp.float32)
    out_ref[...] = (
        p.reshape(NUM_LABELS, TILE_I // CHUNK, CHUNK)
        .transpose(1, 0, 2)
        .reshape(TILE_I // CHUNK * NUM_LABELS, CHUNK))


_mm = pl.pallas_call(
    _mm_body,
    out_shape=jax.ShapeDtypeStruct((MROWS, CHUNK), jnp.float32),
    grid=(GRID,),
    in_specs=[
        pl.BlockSpec((NUM_LABELS, EMBED), lambda b: (0, 0)),
        pl.BlockSpec((EMBED, TILE_I), lambda b: (0, b)),
    ],
    out_specs=pl.BlockSpec((TILE_I // CHUNK * NUM_LABELS, CHUNK),
                           lambda b: (b, 0)),
)


@functools.partial(
    pl.kernel,
    out_type=jax.ShapeDtypeStruct((BATCH,), jnp.float32),
    mesh=_mesh,
    compiler_params=pltpu.CompilerParams(needs_layout_passes=False,
                                         use_tc_tiling_on_sc=True),
    scratch_types=[
        pltpu.VMEM((NCHUNK, CHUNK), jnp.int32),
        pltpu.VMEM((NCHUNK, CHUNK), jnp.int32),
        pltpu.VMEM((1, BPW), jnp.int32),
        pltpu.VMEM((BPW, CHUNK), jnp.float32),
        pltpu.VMEM((BPW,), jnp.float32),
        pltpu.SemaphoreType.DMA,
    ],
)
def _extract(dataT_hbm, mf_hbm, out_hbm,
             idx_v, ridx_v, lbl_v, rows_v, out_v, sem):
    wid = lax.axis_index("s") * 2 + lax.axis_index("c")
    base = wid * BPW

    for c in range(NCHUNK):
        pltpu.sync_copy(
            dataT_hbm.at[pl.ds(0, 1), pl.ds(base + c * CHUNK, CHUNK)],
            idx_v.at[pl.ds(c, 1)])
    pltpu.sync_copy(dataT_hbm.at[pl.ds(1, 1), pl.ds(base, BPW)], lbl_v)
    for c in range(NCHUNK):
        for g in range(CHUNK // LANES):
            w = idx_v[c, pl.ds(g * LANES, LANES)]
            l = lbl_v.at[0][pl.ds(c * CHUNK + g * LANES, LANES)]
            ridx_v[c, pl.ds(g * LANES, LANES)] = ((w >> 7) << 4) + l
    copies = [
        pltpu.async_copy(
            mf_hbm.at[ridx_v.at[c]],
            rows_v.at[pl.ds(c * CHUNK, CHUNK)],
            sem,
        )
        for c in range(NCHUNK)
    ]
    for cp in copies:
        cp.wait()

    iota = lax.iota(jnp.int32, LANES)

    def group(g, carry):
        gbase = pl.multiple_of(g * LANES, LANES)
        cvec = idx_v.at[g // (CHUNK // LANES)][
            pl.ds((g % (CHUNK // LANES)) * LANES, LANES)] & 127
        acc = jnp.zeros((LANES,), jnp.float32)
        for j in range(LANES):
            cj = cvec[j]
            v = rows_v.at[gbase + j][pl.ds((cj >> 4) << 4, LANES)]
            sel = v.at[jnp.full((LANES,), cj & 15, jnp.int32)].get(
                mode="promise_in_bounds")
            acc = jnp.where(iota == j, sel, acc)
        out_v[pl.ds(gbase, LANES)] = acc
        return carry

    lax.fori_loop(0, BPW // LANES, group, 0)

    pltpu.sync_copy(out_v, out_hbm.at[pl.ds(base, BPW)])


def kernel(data, target, word_embedding, label_embedding):
    del target
    mf = _mm(label_embedding, word_embedding.T)
    return _extract(data.astype(jnp.int32).T, mf)


# speedup vs baseline: 9.3682x; 1.0024x over previous
import functools

import jax
import jax.numpy as jnp
from jax import lax
from jax.experimental import pallas as pl
from jax.experimental.pallas import tpu as pltpu
from jax.experimental.pallas import tpu_sc as plsc

BATCH = 16384
EMBED = 64
NUM_LABELS = 16
VOCAB = 1000000
NUM_WORKERS = 32
BPW = BATCH // NUM_WORKERS
CHUNK = 128
NCHUNK = BPW // CHUNK
LANES = 16

TILE_I = 65536
GRID = (VOCAB + TILE_I - 1) // TILE_I
NBLK = (VOCAB + CHUNK - 1) // CHUNK
MROWS = NBLK * NUM_LABELS

_mesh = plsc.VectorSubcoreMesh(core_axis_name="c", subcore_axis_name="s")


def _mm_body(l_ref, wt_ref, out_ref):
    p = lax.dot_general(l_ref[...], wt_ref[...],
                        (((1,), (0,)), ((), ())),
                        preferred_element_type=jnp.float32)
    out_ref[...] = (
        p.reshape(NUM_LABELS, TILE_I // CHUNK, CHUNK)
        .transpose(1, 0, 2)
        .reshape(TILE_I // CHUNK * NUM_LABELS, CHUNK))


_mm = pl.pallas_call(
    _mm_body,
    out_shape=jax.ShapeDtypeStruct((MROWS, CHUNK), jnp.float32),
    grid=(GRID,),
    in_specs=[
        pl.BlockSpec((NUM_LABELS, EMBED), lambda b: (0, 0)),
        pl.BlockSpec((EMBED, TILE_I), lambda b: (0, b)),
    ],
    out_specs=pl.BlockSpec((TILE_I // CHUNK * NUM_LABELS, CHUNK),
                           lambda b: (b, 0)),
)


@functools.partial(
    pl.kernel,
    out_type=jax.ShapeDtypeStruct((BATCH,), jnp.float32),
    mesh=_mesh,
    compiler_params=pltpu.CompilerParams(needs_layout_passes=False,
                                         use_tc_tiling_on_sc=True),
    scratch_types=[
        pltpu.VMEM((NCHUNK, CHUNK), jnp.int32),
        pltpu.VMEM((NCHUNK, CHUNK), jnp.int32),
        pltpu.VMEM((1, BPW), jnp.int32),
        pltpu.VMEM((BPW, CHUNK), jnp.float32),
        pltpu.VMEM((BPW,), jnp.float32),
        pltpu.SemaphoreType.DMA,
    ],
)
def _extract(dataT_hbm, mf_hbm, out_hbm,
             idx_v, ridx_v, lbl_v, rows_v, out_v, sem):
    wid = lax.axis_index("s") * 2 + lax.axis_index("c")
    base = wid * BPW

    for c in range(NCHUNK):
        pltpu.sync_copy(
            dataT_hbm.at[pl.ds(0, 1), pl.ds(base + c * CHUNK, CHUNK)],
            idx_v.at[pl.ds(c, 1)])
    pltpu.sync_copy(dataT_hbm.at[pl.ds(1, 1), pl.ds(base, BPW)], lbl_v)
    for c in range(NCHUNK):
        for g in range(CHUNK // LANES):
            w = idx_v[c, pl.ds(g * LANES, LANES)]
            l = lbl_v.at[0][pl.ds(c * CHUNK + g * LANES, LANES)]
            ridx_v[c, pl.ds(g * LANES, LANES)] = ((w >> 7) << 4) + l
    copies = [
        pltpu.async_copy(
            mf_hbm.at[ridx_v.at[c]],
            rows_v.at[pl.ds(c * CHUNK, CHUNK)],
            sem,
        )
        for c in range(NCHUNK)
    ]
    for cp in copies:
        cp.wait()

    iota = lax.iota(jnp.int32, LANES)

    def group(g, carry):
        gbase = pl.multiple_of(g * LANES, LANES)
        cvec = idx_v.at[g // (CHUNK // LANES)][
            pl.ds((g % (CHUNK // LANES)) * LANES, LANES)] & 127
        acc = jnp.zeros((LANES,), jnp.float32)
        for j in range(LANES):
            cj = cvec[j]
            v = rows_v.at[gbase + j][pl.ds((cj >> 4) << 4, LANES)]
            sel = v.at[jnp.full((LANES,), cj & 15, jnp.int32)].get(
                mode="promise_in_bounds")
            acc = jnp.where(iota == j, sel, acc)
        out_v[pl.ds(gbase, LANES)] = acc
        return carry

    lax.fori_loop(0, BPW // LANES, group, 0)

    pltpu.sync_copy(out_v, out_hbm.at[pl.ds(base, BPW)])


def kernel(data, target, word_embedding, label_embedding):
    del target
    mf = _mm(label_embedding, word_embedding.T)
    return _extract(data.astype(jnp.int32).T, mf)


# speedup vs baseline: 9.3919x; 1.0025x over previous
import functools

import jax
import jax.numpy as jnp
from jax import lax
from jax.experimental import pallas as pl
from jax.experimental.pallas import tpu as pltpu
from jax.experimental.pallas import tpu_sc as plsc

BATCH = 16384
EMBED = 64
NUM_LABELS = 16
VOCAB = 1000000
NUM_WORKERS = 32
BPW = BATCH // NUM_WORKERS
CHUNK = 128
NCHUNK = BPW // CHUNK
LANES = 16

TILE_I = 32768
GRID = (VOCAB + TILE_I - 1) // TILE_I
NBLK = (VOCAB + CHUNK - 1) // CHUNK
MROWS = NBLK * NUM_LABELS

_mesh = plsc.VectorSubcoreMesh(core_axis_name="c", subcore_axis_name="s")


def _mm_body(l_ref, wt_ref, out_ref):
    p = lax.dot_general(l_ref[...], wt_ref[...],
                        (((1,), (0,)), ((), ())),
                        preferred_element_type=jnp.float32)
    out_ref[...] = (
        p.reshape(NUM_LABELS, TILE_I // CHUNK, CHUNK)
        .transpose(1, 0, 2)
        .reshape(TILE_I // CHUNK * NUM_LABELS, CHUNK))


_mm = pl.pallas_call(
    _mm_body,
    out_shape=jax.ShapeDtypeStruct((MROWS, CHUNK), jnp.float32),
    grid=(GRID,),
    in_specs=[
        pl.BlockSpec((NUM_LABELS, EMBED), lambda b: (0, 0)),
        pl.BlockSpec((EMBED, TILE_I), lambda b: (0, b)),
    ],
    out_specs=pl.BlockSpec((TILE_I // CHUNK * NUM_LABELS, CHUNK),
                           lambda b: (b, 0)),
)


@functools.partial(
    pl.kernel,
    out_type=jax.ShapeDtypeStruct((BATCH,), jnp.float32),
    mesh=_mesh,
    compiler_params=pltpu.CompilerParams(needs_layout_passes=False,
                                         use_tc_tiling_on_sc=True),
    scratch_types=[
        pltpu.VMEM((NCHUNK, CHUNK), jnp.int32),
        pltpu.VMEM((NCHUNK, CHUNK), jnp.int32),
        pltpu.VMEM((1, BPW), jnp.int32),
        pltpu.VMEM((BPW, CHUNK), jnp.float32),
        pltpu.VMEM((BPW,), jnp.float32),
        pltpu.SemaphoreType.DMA,
    ],
)
def _extract(dataT_hbm, mf_hbm, out_hbm,
             idx_v, ridx_v, lbl_v, rows_v, out_v, sem):
    wid = lax.axis_index("s") * 2 + lax.axis_index("c")
    base = wid * BPW

    for c in range(NCHUNK):
        pltpu.sync_copy(
            dataT_hbm.at[pl.ds(0, 1), pl.ds(base + c * CHUNK, CHUNK)],
            idx_v.at[pl.ds(c, 1)])
    pltpu.sync_copy(dataT_hbm.at[pl.ds(1, 1), pl.ds(base, BPW)], lbl_v)
    for c in range(NCHUNK):
        for g in range(CHUNK // LANES):
            w = idx_v[c, pl.ds(g * LANES, LANES)]
            l = lbl_v.at[0][pl.ds(c * CHUNK + g * LANES, LANES)]
            ridx_v[c, pl.ds(g * LANES, LANES)] = ((w >> 7) << 4) + l
    copies = [
        pltpu.async_copy(
            mf_hbm.at[ridx_v.at[c]],
            rows_v.at[pl.ds(c * CHUNK, CHUNK)],
            sem,
        )
        for c in range(NCHUNK)
    ]
    for cp in copies:
        cp.wait()

    iota = lax.iota(jnp.int32, LANES)

    def group(g, carry):
        gbase = pl.multiple_of(g * LANES, LANES)
        cvec = idx_v.at[g // (CHUNK // LANES)][
            pl.ds((g % (CHUNK // LANES)) * LANES, LANES)] & 127
        acc = jnp.zeros((LANES,), jnp.float32)
        for j in range(LANES):
            cj = cvec[j]
            v = rows_v.at[gbase + j][pl.ds((cj >> 4) << 4, LANES)]
            sel = v.at[jnp.full((LANES,), cj & 15, jnp.int32)].get(
                mode="promise_in_bounds")
            acc = jnp.where(iota == j, sel, acc)
        out_v[pl.ds(gbase, LANES)] = acc
        return carry

    lax.fori_loop(0, BPW // LANES, group, 0)

    pltpu.sync_copy(out_v, out_hbm.at[pl.ds(base, BPW)])


def kernel(data, target, word_embedding, label_embedding):
    del target
    mf = _mm(label_embedding, word_embedding.T)
    return _extract(data.astype(jnp.int32).T, mf)
